# baseline (speedup 1.0000x reference)
import functools

import jax
import jax.numpy as jnp
from jax import lax
from jax.experimental import pallas as pl
from jax.experimental.pallas import tpu as pltpu
from jax.experimental.pallas import tpu_sc as plsc

EMBED_DIM = 64
NUM_ROWS = 16384
NUM_COLS = 200
B = NUM_ROWS * NUM_COLS

NC = 2
NS = 16
NW = NC * NS
BPW = B // NW

CHUNK = 1024
SUB = 128
K = CHUNK // SUB
NGROUPS = BPW // CHUNK


@functools.partial(
    pl.kernel,
    mesh=plsc.VectorSubcoreMesh(core_axis_name="c", subcore_axis_name="s"),
    out_type=jax.ShapeDtypeStruct((B, EMBED_DIM), jnp.float32),
    scratch_types=[
        pltpu.VMEM((CHUNK,), jnp.int32),
        pltpu.VMEM((CHUNK, EMBED_DIM), jnp.float32),
        pltpu.SemaphoreType.DMA,
    ],
    compiler_params=pltpu.CompilerParams(use_tc_tiling_on_sc=False),
)
def _sc_gather(idx_hbm, table_hbm, out_hbm, idx_v, rows_v, gsem):
    wid = lax.axis_index("s") * NC + lax.axis_index("c")
    base = wid * BPW

    def group(g, carry):
        off = base + g * CHUNK
        pltpu.sync_copy(idx_hbm.at[pl.ds(off, CHUNK)], idx_v)
        copies = [
            pltpu.async_copy(
                table_hbm.at[idx_v.at[pl.ds(j * SUB, SUB)]],
                rows_v.at[pl.ds(j * SUB, SUB)],
                gsem,
            )
            for j in range(K)
        ]
        for cp in copies:
            cp.wait()
        pltpu.sync_copy(rows_v, out_hbm.at[pl.ds(off, CHUNK)])
        return carry

    lax.fori_loop(0, NGROUPS, group, 0)


def kernel(data, embedding):
    flat_idx = data.reshape(B).astype(jnp.int32)
    out = _sc_gather(flat_idx, embedding)
    return out.reshape(NUM_ROWS, NUM_COLS, EMBED_DIM)


# speedup vs baseline: 1.0318x; 1.0318x over previous
import functools

import jax
import jax.numpy as jnp
from jax import lax
from jax.experimental import pallas as pl
from jax.experimental.pallas import tpu as pltpu
from jax.experimental.pallas import tpu_sc as plsc

EMBED_DIM = 64
NUM_ROWS = 16384
NUM_COLS = 200
B = NUM_ROWS * NUM_COLS

NC = 2
NS = 16
NW = NC * NS
BPW = B // NW

CHUNK = 640
SUB = 128
K = CHUNK // SUB
NGROUPS = BPW // CHUNK


@functools.partial(
    pl.kernel,
    mesh=plsc.VectorSubcoreMesh(core_axis_name="c", subcore_axis_name="s"),
    out_type=jax.ShapeDtypeStruct((B, EMBED_DIM), jnp.float32),
    scratch_types=[
        pltpu.VMEM((CHUNK,), jnp.int32),
        pltpu.VMEM((CHUNK,), jnp.int32),
        pltpu.VMEM((CHUNK, EMBED_DIM), jnp.float32),
        pltpu.VMEM((CHUNK, EMBED_DIM), jnp.float32),
        pltpu.SemaphoreType.DMA,
        pltpu.SemaphoreType.DMA,
        pltpu.SemaphoreType.DMA,
        pltpu.SemaphoreType.DMA,
        pltpu.SemaphoreType.DMA,
        pltpu.SemaphoreType.DMA,
    ],
    compiler_params=pltpu.CompilerParams(use_tc_tiling_on_sc=False),
)
def _sc_gather(idx_hbm, table_hbm, out_hbm, idx0, idx1, rows0, rows1,
               isem0, isem1, gsem0, gsem1, ssem0, ssem1):
    wid = lax.axis_index("s") * NC + lax.axis_index("c")
    base = wid * BPW

    idx_b = (idx0, idx1)
    rows_b = (rows0, rows1)
    isem_b = (isem0, isem1)
    gsem_b = (gsem0, gsem1)
    ssem_b = (ssem0, ssem1)

    def idx_slice(g):
        return idx_hbm.at[pl.ds(base + g * CHUNK, CHUNK)]

    def out_slice(g):
        return out_hbm.at[pl.ds(base + g * CHUNK, CHUNK)]

    pltpu.async_copy(idx_slice(0), idx0, isem0)
    pltpu.async_copy(idx_slice(1), idx1, isem1)

    def pair(gp, carry):
        for b in range(2):
            g = gp * 2 + b
            pltpu.make_async_copy(idx_slice(g), idx_b[b], isem_b[b]).wait()
            @pl.when(g >= 2)
            def _():
                pltpu.make_async_copy(rows_b[b], out_slice(g - 2),
                                      ssem_b[b]).wait()
            copies = [
                pltpu.async_copy(
                    table_hbm.at[idx_b[b].at[pl.ds(j * SUB, SUB)]],
                    rows_b[b].at[pl.ds(j * SUB, SUB)],
                    gsem_b[b],
                )
                for j in range(K)
            ]
            for cp in copies:
                cp.wait()
            @pl.when(g + 2 < NGROUPS)
            def _():
                pltpu.async_copy(idx_slice(g + 2), idx_b[b], isem_b[b])
            pltpu.async_copy(rows_b[b], out_slice(g), ssem_b[b])
        return carry

    lax.fori_loop(0, NGROUPS // 2, pair, 0)

    pltpu.make_async_copy(rows0, out_slice(NGROUPS - 2), ssem0).wait()
    pltpu.make_async_copy(rows1, out_slice(NGROUPS - 1), ssem1).wait()


def kernel(data, embedding):
    flat_idx = data.reshape(B).astype(jnp.int32)
    out = _sc_gather(flat_idx, embedding)
    return out.reshape(NUM_ROWS, NUM_COLS, EMBED_DIM)
